# baseline (speedup 1.0000x reference)
import functools
import math

import jax
import jax.numpy as jnp
from jax import lax
from jax.experimental import pallas as pl
from jax.experimental.pallas import tpu as pltpu
from jax.experimental.pallas import tpu_sc as plsc

_LANES = 16
_NUM_WORKERS = 32


@functools.cache
def _make_sc_kernel(total_rows, batch, d_model, chunk, nbuf):
    assert total_rows % (_NUM_WORKERS * chunk) == 0
    assert chunk % batch == 0 and d_model % _LANES == 0
    rows_per_worker = total_rows // _NUM_WORKERS
    n_chunks = rows_per_worker // chunk
    assert n_chunks % nbuf == 0 and n_chunks >= nbuf
    pe_rows = chunk // batch
    nvec = d_model // _LANES
    scale = math.sqrt(d_model)

    mesh = plsc.VectorSubcoreMesh(core_axis_name="c", subcore_axis_name="s")

    @functools.partial(
        pl.kernel,
        mesh=mesh,
        out_type=jax.ShapeDtypeStruct((total_rows, d_model), jnp.float32),
        scratch_types=[
            pltpu.VMEM((rows_per_worker,), jnp.int32),
            pltpu.VMEM((nbuf, chunk, d_model), jnp.float32),
            pltpu.VMEM((nbuf, pe_rows, d_model), jnp.float32),
            pltpu.SemaphoreType.DMA((nbuf,)),
            pltpu.SemaphoreType.DMA((nbuf,)),
            pltpu.SemaphoreType.DMA((nbuf,)),
        ],
    )
    def sc_kernel(idx_hbm, table_hbm, pe_hbm, out_hbm, idx_v, rows_v, pe_v,
                  gsem, psem, ssem):
        wid = lax.axis_index("s") * 2 + lax.axis_index("c")
        base = pl.multiple_of(wid * rows_per_worker, rows_per_worker)
        pltpu.sync_copy(idx_hbm.at[pl.ds(base, rows_per_worker)], idx_v)
        pe_base = base // batch

        def start_gather(c, b):
            off = pl.multiple_of(c * chunk, chunk)
            pltpu.async_copy(
                table_hbm.at[idx_v.at[pl.ds(off, chunk)]],
                rows_v.at[b], gsem.at[b])
            pb = pl.multiple_of(pe_base + c * pe_rows, pe_rows)
            pltpu.async_copy(pe_hbm.at[pl.ds(pb, pe_rows)], pe_v.at[b],
                             psem.at[b])

        def wait_gather(b):
            pltpu.make_async_copy(
                table_hbm.at[idx_v.at[pl.ds(0, chunk)]],
                rows_v.at[b], gsem.at[b]).wait()
            pltpu.make_async_copy(pe_hbm.at[pl.ds(0, pe_rows)], pe_v.at[b],
                                  psem.at[b]).wait()

        def start_store(c, b):
            off = pl.multiple_of(base + c * chunk, chunk)
            pltpu.async_copy(rows_v.at[b], out_hbm.at[pl.ds(off, chunk)],
                             ssem.at[b])

        def wait_store(b):
            pltpu.make_async_copy(rows_v.at[b], out_hbm.at[pl.ds(0, chunk)],
                                  ssem.at[b]).wait()

        def compute(b):
            def grp_body(g, _):
                row0 = g * batch
                for k in range(nvec):
                    sl = pl.ds(k * _LANES, _LANES)
                    p = pe_v[b, g, sl]
                    for r in range(batch):
                        rows_v[b, row0 + r, sl] = (
                            rows_v[b, row0 + r, sl] * scale + p)
                return 0

            lax.fori_loop(0, pe_rows, grp_body, 0)

        for b in range(nbuf - 1):
            start_gather(b, b)

        def group_body(i, _):
            c0 = i * nbuf
            for b in range(nbuf):
                c = c0 + b
                wait_gather(b)
                compute(b)
                start_store(c, b)
                cp = c + nbuf - 1
                bp = (b + nbuf - 1) % nbuf

                @pl.when(cp < n_chunks)
                def _():
                    @pl.when(c >= 1)
                    def _():
                        wait_store(bp)
                    start_gather(cp, bp)
            return 0

        lax.fori_loop(0, n_chunks // nbuf, group_body, 0)
        for b in range(nbuf):
            wait_store(b)

    return sc_kernel


def kernel(x, table, pos_encodings):
    seq_len, batch = x.shape
    _, d_model = table.shape
    x_flat = x.reshape(seq_len * batch).astype(jnp.int32)
    pe = pos_encodings.reshape(pos_encodings.shape[0], d_model)
    fn = _make_sc_kernel(seq_len * batch, batch, d_model, 16, 4)
    out = fn(x_flat, table, pe)
    return out.reshape(seq_len, batch, d_model)


# speedup vs baseline: 1.5648x; 1.5648x over previous
import functools
import math

import jax
import jax.numpy as jnp
from jax import lax
from jax.experimental import pallas as pl
from jax.experimental.pallas import tpu as pltpu
from jax.experimental.pallas import tpu_sc as plsc

_LANES = 16
_NUM_WORKERS = 32


@functools.cache
def _make_sc_kernel(seq_len, batch, d_model, max_len, chunk, nbuf):
    total_rows = seq_len * batch
    assert total_rows % (_NUM_WORKERS * chunk) == 0
    assert chunk % batch == 0 and d_model % _LANES == 0
    rows_per_worker = total_rows // _NUM_WORKERS
    n_chunks = rows_per_worker // chunk
    assert n_chunks % nbuf == 0 and n_chunks >= nbuf
    pe_rows = chunk // batch
    nvec = d_model // _LANES
    scale = math.sqrt(d_model)

    mesh = plsc.VectorSubcoreMesh(core_axis_name="c", subcore_axis_name="s")

    @functools.partial(
        pl.kernel,
        mesh=mesh,
        out_type=jax.ShapeDtypeStruct((seq_len, batch, d_model), jnp.float32),
        scratch_types=[
            pltpu.VMEM((rows_per_worker,), jnp.int32),
            pltpu.VMEM((nbuf, pe_rows, batch, d_model), jnp.float32),
            pltpu.VMEM((nbuf, pe_rows, 1, d_model), jnp.float32),
            pltpu.SemaphoreType.DMA((nbuf,)),
            pltpu.SemaphoreType.DMA((nbuf,)),
            pltpu.SemaphoreType.DMA((nbuf,)),
        ],
    )
    def sc_kernel(idx_hbm, table_hbm, pe_hbm, out_hbm, idx_v, rows_v, pe_v,
                  gsem, psem, ssem):
        wid = lax.axis_index("s") * 2 + lax.axis_index("c")
        base = pl.multiple_of(wid * rows_per_worker, rows_per_worker)
        pltpu.sync_copy(idx_hbm.at[pl.ds(base, rows_per_worker)], idx_v)
        seq_base = base // batch

        def start_gather(c, b):
            off = pl.multiple_of(c * chunk, chunk)
            pltpu.async_copy(
                table_hbm.at[idx_v.at[pl.ds(off, chunk)]],
                rows_v.at[b].reshape(chunk, d_model), gsem.at[b])
            pb = pl.multiple_of(seq_base + c * pe_rows, pe_rows)
            pltpu.async_copy(pe_hbm.at[pl.ds(pb, pe_rows)], pe_v.at[b],
                             psem.at[b])

        def wait_gather(b):
            pltpu.make_async_copy(
                table_hbm.at[idx_v.at[pl.ds(0, chunk)]],
                rows_v.at[b].reshape(chunk, d_model), gsem.at[b]).wait()
            pltpu.make_async_copy(pe_hbm.at[pl.ds(0, pe_rows)], pe_v.at[b],
                                  psem.at[b]).wait()

        def start_store(c, b):
            sb = pl.multiple_of(seq_base + c * pe_rows, pe_rows)
            pltpu.async_copy(rows_v.at[b], out_hbm.at[pl.ds(sb, pe_rows)],
                             ssem.at[b])

        def wait_store(b):
            pltpu.make_async_copy(rows_v.at[b], out_hbm.at[pl.ds(0, pe_rows)],
                                  ssem.at[b]).wait()

        def compute(b):
            @plsc.parallel_loop(0, pe_rows, step=1, unroll=2)
            def grp_body(g):
                for k in range(nvec):
                    sl = pl.ds(k * _LANES, _LANES)
                    p = pe_v[b, g, 0, sl]
                    for r in range(batch):
                        rows_v[b, g, r, sl] = rows_v[b, g, r, sl] * scale + p

        for b in range(nbuf - 1):
            start_gather(b, b)

        def group_body(i, _):
            c0 = i * nbuf
            for b in range(nbuf):
                c = c0 + b
                wait_gather(b)
                compute(b)
                start_store(c, b)
                cp = c + nbuf - 1
                bp = (b + nbuf - 1) % nbuf

                @pl.when(cp < n_chunks)
                def _():
                    @pl.when(c >= 1)
                    def _():
                        wait_store(bp)
                    start_gather(cp, bp)
            return 0

        lax.fori_loop(0, n_chunks // nbuf, group_body, 0)
        for b in range(nbuf):
            wait_store(b)

    return sc_kernel


def kernel(x, table, pos_encodings):
    seq_len, batch = x.shape
    _, d_model = table.shape
    max_len = pos_encodings.shape[0]
    x_flat = x.reshape(seq_len * batch).astype(jnp.int32)
    fn = _make_sc_kernel(seq_len, batch, d_model, max_len, 16, 4)
    return fn(x_flat, table, pos_encodings)
